# baseline (speedup 1.0000x reference)
import functools

import jax
import jax.numpy as jnp
from jax import lax
from jax.experimental import pallas as pl
from jax.experimental.pallas import tpu as pltpu
from jax.experimental.pallas import tpu_sc as plsc


BATCH = 4
SEQ_LEN = 4096
D_MODEL = 1024

NUM_CORES = 2
NUM_SUBCORES = 16
NUM_WORKERS = NUM_CORES * NUM_SUBCORES
ROWS_PER_WORKER = SEQ_LEN // NUM_WORKERS
CHUNK = 32
CHUNKS_PER_WORKER = ROWS_PER_WORKER // CHUNK
LANES = 16
COL_GROUPS = D_MODEL // LANES


def _sc_body(pos_hbm, x_hbm, out_hbm, tbuf, xbuf):
    c = lax.axis_index("c")
    s = lax.axis_index("s")
    wid = s * NUM_CORES + c

    def chunk_body(k, _):
        trow = wid * ROWS_PER_WORKER + k * CHUNK
        pltpu.sync_copy(pos_hbm.at[pl.ds(trow, CHUNK)], tbuf)

        def batch_body(b, _):
            xrow = b * SEQ_LEN + trow
            pltpu.sync_copy(x_hbm.at[pl.ds(xrow, CHUNK)], xbuf)

            def row_body(r, _):
                for cc in range(COL_GROUPS):
                    v = tbuf[r, pl.ds(cc * LANES, LANES)]
                    plsc.addupdate(xbuf.at[r, pl.ds(cc * LANES, LANES)], v)
                return 0

            lax.fori_loop(0, CHUNK, row_body, 0)
            pltpu.sync_copy(xbuf, out_hbm.at[pl.ds(xrow, CHUNK)])
            return 0

        lax.fori_loop(0, BATCH, batch_body, 0)
        return 0

    lax.fori_loop(0, CHUNKS_PER_WORKER, chunk_body, 0)


def kernel(x, pos_table):
    xf = x.reshape(BATCH * SEQ_LEN, D_MODEL)
    mesh = plsc.VectorSubcoreMesh(core_axis_name="c", subcore_axis_name="s")
    out = pl.kernel(
        _sc_body,
        out_type=jax.ShapeDtypeStruct((BATCH * SEQ_LEN, D_MODEL), x.dtype),
        mesh=mesh,
        scratch_types=[
            pltpu.VMEM((CHUNK, D_MODEL), jnp.float32),
            pltpu.VMEM((CHUNK, D_MODEL), jnp.float32),
        ],
    )(pos_table, xf)
    return out.reshape(BATCH, SEQ_LEN, D_MODEL)
